# Initial kernel scaffold; baseline (speedup 1.0000x reference)
import jax
import jax.numpy as jnp
from jax.experimental import pallas as pl


def kernel(history_data, future_data, batch_seen, epoch, train, node_emb, time_in_day_emb, day_in_week_emb, ts_W, ts_b, fc1_W_0, fc1_b_0, fc2_W_0, fc2_b_0, fc1_W_1, fc1_b_1, fc2_W_1, fc2_b_1, reg_W, reg_b):
    raise NotImplementedError("write your pallas kernel here")


# speedup vs baseline: 5.0101x; 5.0101x over previous
import jax
import jax.numpy as jnp
from jax import lax
from jax.experimental import pallas as pl
from jax.experimental.pallas import tpu as pltpu

_B, _L, _N, _C = 32, 12, 10000, 3
_EMBED_DIM = 32
_NODE_DIM = 16
_TID_DIM = 16
_DIW_DIM = 16
_TOD_SIZE = 288
_DOW_SIZE = 7
_OUTPUT_LEN = 12
_HIDDEN = 80

_NBLK = 2048
_NB = (_N + _NBLK - 1) // _NBLK


def _minmax_kernel(d_ref, mn_ref, mx_ref):
    b = pl.program_id(0)
    cur_mn = jnp.min(d_ref[...])
    cur_mx = jnp.max(d_ref[...])

    @pl.when(b == 0)
    def _init():
        mn_ref[0, 0] = cur_mn
        mx_ref[0, 0] = cur_mx

    @pl.when(b != 0)
    def _acc():
        mn_ref[0, 0] = jnp.minimum(mn_ref[0, 0], cur_mn)
        mx_ref[0, 0] = jnp.maximum(mx_ref[0, 0], cur_mx)


def _stid_kernel(mn_ref, mx_ref, x_ref, tid_ref, diw_ref, node_ref,
                 ttab_ref, dtab_ref, tsW_ref, tsb_ref,
                 w10_ref, b10_ref, w20_ref, b20_ref,
                 w11_ref, b11_ref, w21_ref, b21_ref,
                 regW_ref, regb_ref, out_ref):
    f32 = jnp.float32
    x = x_ref[0]
    ts = jnp.dot(tsW_ref[...], x, preferred_element_type=f32) + tsb_ref[...]

    tid_f = tid_ref[0]
    tid_f = tid_f - jnp.floor(tid_f / _TOD_SIZE) * _TOD_SIZE
    tid_idx = tid_f.astype(jnp.int32)
    oh_t = (lax.broadcasted_iota(jnp.int32, (_TOD_SIZE, x.shape[1]), 0)
            == tid_idx).astype(f32)
    tid_e = jnp.dot(ttab_ref[...], oh_t, preferred_element_type=f32)
    tid_e = tid_e + jnp.where(tid_idx >= _TOD_SIZE, jnp.nan, 0.0)

    mn = mn_ref[0, 0]
    shift_max = mx_ref[0, 0] - mn
    dn = (diw_ref[0] - mn) / (shift_max + 1e-8)
    diw_idx = jnp.clip((dn * _DOW_SIZE).astype(jnp.int32), 0, _DOW_SIZE - 1)
    oh_d = (lax.broadcasted_iota(jnp.int32, (8, x.shape[1]), 0)
            == diw_idx).astype(f32)
    diw_e = jnp.dot(dtab_ref[...], oh_d, preferred_element_type=f32)

    h = jnp.concatenate([ts, node_ref[...], tid_e, diw_e], axis=0)

    for w_ref, b_ref, w2_ref, b2_ref in ((w10_ref, b10_ref, w20_ref, b20_ref),
                                         (w11_ref, b11_ref, w21_ref, b21_ref)):
        t = jnp.dot(w_ref[...], h, preferred_element_type=f32) + b_ref[...]
        t = jnp.maximum(t, 0.0)
        t = jnp.dot(w2_ref[...], t, preferred_element_type=f32) + b2_ref[...]
        h = h + t

    out_ref[0] = jnp.dot(regW_ref[...], h, preferred_element_type=f32) + regb_ref[...]


def _run(history_data, node_emb,
         time_in_day_emb, day_in_week_emb, ts_W, ts_b,
         fc1_W_0, fc1_b_0, fc2_W_0, fc2_b_0,
         fc1_W_1, fc1_b_1, fc2_W_1, fc2_b_1, reg_W, reg_b):
    f32 = jnp.float32
    x_in = history_data[..., 0]
    tid_last = history_data[:, -1, :, 1].reshape(_B, 1, _N)
    diw_full = history_data[..., 2]
    diw_last = history_data[:, -1, :, 2].reshape(_B, 1, _N)
    node_T = jnp.transpose(node_emb)
    ttab_T = jnp.transpose(time_in_day_emb)
    dtab_T = jnp.pad(jnp.transpose(day_in_week_emb), ((0, 0), (0, 1)))

    mn, mx = pl.pallas_call(
        _minmax_kernel,
        grid=(_B,),
        in_specs=[pl.BlockSpec((1, _L, _N), lambda b: (b, 0, 0))],
        out_specs=[
            pl.BlockSpec((1, 1), lambda b: (0, 0), memory_space=pltpu.SMEM),
            pl.BlockSpec((1, 1), lambda b: (0, 0), memory_space=pltpu.SMEM),
        ],
        out_shape=[
            jax.ShapeDtypeStruct((1, 1), f32),
            jax.ShapeDtypeStruct((1, 1), f32),
        ],
    )(diw_full)

    def spec_const(shape, space=None):
        if space is None:
            return pl.BlockSpec(shape, lambda i, j: tuple(0 for _ in shape))
        return pl.BlockSpec(shape, lambda i, j: tuple(0 for _ in shape),
                            memory_space=space)

    out = pl.pallas_call(
        _stid_kernel,
        grid=(_NB, _B),
        in_specs=[
            spec_const((1, 1), pltpu.SMEM),
            spec_const((1, 1), pltpu.SMEM),
            pl.BlockSpec((1, _L, _NBLK), lambda i, j: (j, 0, i)),
            pl.BlockSpec((1, 1, _NBLK), lambda i, j: (j, 0, i)),
            pl.BlockSpec((1, 1, _NBLK), lambda i, j: (j, 0, i)),
            pl.BlockSpec((_NODE_DIM, _NBLK), lambda i, j: (0, i)),
            spec_const((_TID_DIM, _TOD_SIZE)),
            spec_const((_DIW_DIM, 8)),
            spec_const((_EMBED_DIM, _L)),
            spec_const((_EMBED_DIM, 1)),
            spec_const((_HIDDEN, _HIDDEN)),
            spec_const((_HIDDEN, 1)),
            spec_const((_HIDDEN, _HIDDEN)),
            spec_const((_HIDDEN, 1)),
            spec_const((_HIDDEN, _HIDDEN)),
            spec_const((_HIDDEN, 1)),
            spec_const((_HIDDEN, _HIDDEN)),
            spec_const((_HIDDEN, 1)),
            spec_const((_OUTPUT_LEN, _HIDDEN)),
            spec_const((_OUTPUT_LEN, 1)),
        ],
        out_specs=pl.BlockSpec((1, _OUTPUT_LEN, _NBLK), lambda i, j: (j, 0, i)),
        out_shape=jax.ShapeDtypeStruct((_B, _OUTPUT_LEN, _N), f32),
    )(mn, mx, x_in, tid_last, diw_last, node_T, ttab_T, dtab_T,
      ts_W, ts_b.reshape(_EMBED_DIM, 1),
      fc1_W_0, fc1_b_0.reshape(_HIDDEN, 1), fc2_W_0, fc2_b_0.reshape(_HIDDEN, 1),
      fc1_W_1, fc1_b_1.reshape(_HIDDEN, 1), fc2_W_1, fc2_b_1.reshape(_HIDDEN, 1),
      reg_W, reg_b.reshape(_OUTPUT_LEN, 1))
    return out[..., None]


def kernel(history_data, future_data, batch_seen, epoch, train,
           node_emb, time_in_day_emb, day_in_week_emb, ts_W, ts_b,
           fc1_W_0, fc1_b_0, fc2_W_0, fc2_b_0,
           fc1_W_1, fc1_b_1, fc2_W_1, fc2_b_1,
           reg_W, reg_b):
    del future_data, batch_seen, epoch, train
    return _run(history_data, node_emb,
                time_in_day_emb, day_in_week_emb, ts_W, ts_b,
                fc1_W_0, fc1_b_0, fc2_W_0, fc2_b_0,
                fc1_W_1, fc1_b_1, fc2_W_1, fc2_b_1, reg_W, reg_b)


# speedup vs baseline: 5.1266x; 1.0233x over previous
import jax
import jax.numpy as jnp
from jax import lax
from jax.experimental import pallas as pl
from jax.experimental.pallas import tpu as pltpu

_B, _L, _N, _C = 32, 12, 10000, 3
_EMBED_DIM = 32
_NODE_DIM = 16
_TID_DIM = 16
_DIW_DIM = 16
_TOD_SIZE = 288
_DOW_SIZE = 7
_OUTPUT_LEN = 12
_HIDDEN = 80

_NBLK = 2048
_NB = (_N + _NBLK - 1) // _NBLK


def _minmax_kernel(d_ref, mn_ref, mx_ref):
    b = pl.program_id(0)
    cur_mn = jnp.min(d_ref[...])
    cur_mx = jnp.max(d_ref[...])

    @pl.when(b == 0)
    def _init():
        mn_ref[0, 0] = cur_mn
        mx_ref[0, 0] = cur_mx

    @pl.when(b != 0)
    def _acc():
        mn_ref[0, 0] = jnp.minimum(mn_ref[0, 0], cur_mn)
        mx_ref[0, 0] = jnp.maximum(mx_ref[0, 0], cur_mx)


def _stid_kernel(mn_ref, mx_ref, x_ref, tid_ref, diw_ref, node_ref,
                 ttab_ref, dtab_ref, tsW_ref, tsb_ref,
                 w10_ref, b10_ref, w20_ref, b20_ref,
                 w11_ref, b11_ref, w21_ref, b21_ref,
                 regW_ref, regb_ref, out_ref):
    f32 = jnp.float32
    bf16 = jnp.bfloat16
    x = x_ref[0]
    ts = jnp.dot(tsW_ref[...], x, preferred_element_type=f32) + tsb_ref[...]

    tid_f = tid_ref[0]
    tid_f = tid_f - jnp.floor(tid_f / _TOD_SIZE) * _TOD_SIZE
    tid_idx = tid_f.astype(jnp.int32)
    oh_t = (lax.broadcasted_iota(jnp.int32, (_TOD_SIZE, tid_idx.shape[1]), 0)
            == tid_idx).astype(bf16)
    tid_e = jnp.dot(ttab_ref[...], oh_t, preferred_element_type=f32)
    tid_e = tid_e + jnp.where(tid_idx >= _TOD_SIZE, jnp.nan, 0.0)

    mn = mn_ref[0, 0]
    shift_max = mx_ref[0, 0] - mn
    dn = (diw_ref[0] - mn) / (shift_max + 1e-8)
    diw_idx = jnp.clip((dn * _DOW_SIZE).astype(jnp.int32), 0, _DOW_SIZE - 1)
    oh_d = (lax.broadcasted_iota(jnp.int32, (8, diw_idx.shape[1]), 0)
            == diw_idx).astype(bf16)
    diw_e = jnp.dot(dtab_ref[...], oh_d, preferred_element_type=f32)

    h = jnp.concatenate([ts, node_ref[...], tid_e, diw_e], axis=0)

    for w_ref, b_ref, w2_ref, b2_ref in ((w10_ref, b10_ref, w20_ref, b20_ref),
                                         (w11_ref, b11_ref, w21_ref, b21_ref)):
        t = jnp.dot(w_ref[...], h.astype(bf16),
                    preferred_element_type=f32) + b_ref[...]
        t = jnp.maximum(t, 0.0)
        t = jnp.dot(w2_ref[...], t.astype(bf16),
                    preferred_element_type=f32) + b2_ref[...]
        h = h + t

    out_ref[0] = jnp.dot(regW_ref[...], h.astype(bf16),
                         preferred_element_type=f32) + regb_ref[...]


def _run(history_data, node_emb,
         time_in_day_emb, day_in_week_emb, ts_W, ts_b,
         fc1_W_0, fc1_b_0, fc2_W_0, fc2_b_0,
         fc1_W_1, fc1_b_1, fc2_W_1, fc2_b_1, reg_W, reg_b):
    f32 = jnp.float32
    bf16 = jnp.bfloat16
    x_in = history_data[..., 0].astype(bf16)
    tid_last = history_data[:, -1, :, 1].reshape(_B, 1, _N)
    diw_full = history_data[..., 2]
    diw_last = history_data[:, -1, :, 2].reshape(_B, 1, _N)
    node_T = jnp.transpose(node_emb)
    ttab_T = jnp.transpose(time_in_day_emb).astype(bf16)
    dtab_T = jnp.pad(jnp.transpose(day_in_week_emb), ((0, 0), (0, 1))).astype(bf16)

    mn, mx = pl.pallas_call(
        _minmax_kernel,
        grid=(_B,),
        in_specs=[pl.BlockSpec((1, _L, _N), lambda b: (b, 0, 0))],
        out_specs=[
            pl.BlockSpec((1, 1), lambda b: (0, 0), memory_space=pltpu.SMEM),
            pl.BlockSpec((1, 1), lambda b: (0, 0), memory_space=pltpu.SMEM),
        ],
        out_shape=[
            jax.ShapeDtypeStruct((1, 1), f32),
            jax.ShapeDtypeStruct((1, 1), f32),
        ],
    )(diw_full)

    def spec_const(shape, space=None):
        if space is None:
            return pl.BlockSpec(shape, lambda i, j: tuple(0 for _ in shape))
        return pl.BlockSpec(shape, lambda i, j: tuple(0 for _ in shape),
                            memory_space=space)

    out = pl.pallas_call(
        _stid_kernel,
        grid=(_NB, _B),
        in_specs=[
            spec_const((1, 1), pltpu.SMEM),
            spec_const((1, 1), pltpu.SMEM),
            pl.BlockSpec((1, _L, _NBLK), lambda i, j: (j, 0, i)),
            pl.BlockSpec((1, 1, _NBLK), lambda i, j: (j, 0, i)),
            pl.BlockSpec((1, 1, _NBLK), lambda i, j: (j, 0, i)),
            pl.BlockSpec((_NODE_DIM, _NBLK), lambda i, j: (0, i)),
            spec_const((_TID_DIM, _TOD_SIZE)),
            spec_const((_DIW_DIM, 8)),
            spec_const((_EMBED_DIM, _L)),
            spec_const((_EMBED_DIM, 1)),
            spec_const((_HIDDEN, _HIDDEN)),
            spec_const((_HIDDEN, 1)),
            spec_const((_HIDDEN, _HIDDEN)),
            spec_const((_HIDDEN, 1)),
            spec_const((_HIDDEN, _HIDDEN)),
            spec_const((_HIDDEN, 1)),
            spec_const((_HIDDEN, _HIDDEN)),
            spec_const((_HIDDEN, 1)),
            spec_const((_OUTPUT_LEN, _HIDDEN)),
            spec_const((_OUTPUT_LEN, 1)),
        ],
        out_specs=pl.BlockSpec((1, _OUTPUT_LEN, _NBLK), lambda i, j: (j, 0, i)),
        out_shape=jax.ShapeDtypeStruct((_B, _OUTPUT_LEN, _N), f32),
    )(mn, mx, x_in, tid_last, diw_last, node_T, ttab_T, dtab_T,
      ts_W.astype(bf16), ts_b.reshape(_EMBED_DIM, 1),
      fc1_W_0.astype(bf16), fc1_b_0.reshape(_HIDDEN, 1),
      fc2_W_0.astype(bf16), fc2_b_0.reshape(_HIDDEN, 1),
      fc1_W_1.astype(bf16), fc1_b_1.reshape(_HIDDEN, 1),
      fc2_W_1.astype(bf16), fc2_b_1.reshape(_HIDDEN, 1),
      reg_W.astype(bf16), reg_b.reshape(_OUTPUT_LEN, 1))
    return out[..., None]


def kernel(history_data, future_data, batch_seen, epoch, train,
           node_emb, time_in_day_emb, day_in_week_emb, ts_W, ts_b,
           fc1_W_0, fc1_b_0, fc2_W_0, fc2_b_0,
           fc1_W_1, fc1_b_1, fc2_W_1, fc2_b_1,
           reg_W, reg_b):
    del future_data, batch_seen, epoch, train
    return _run(history_data, node_emb,
                time_in_day_emb, day_in_week_emb, ts_W, ts_b,
                fc1_W_0, fc1_b_0, fc2_W_0, fc2_b_0,
                fc1_W_1, fc1_b_1, fc2_W_1, fc2_b_1, reg_W, reg_b)
